# baseline (speedup 1.0000x reference)
import functools

import jax
import jax.numpy as jnp
from jax import lax
from jax.experimental import pallas as pl
from jax.experimental.pallas import tpu as pltpu
from jax.experimental.pallas import tpu_sc as plsc

N = 10000
D = 128
H = 64
G = 64

NC = 2
NS = 16
NW = NC * NS
C = 128
NCH = 80
EPT = NCH * C
E_PAD = EPT * NW
NB = 8

CH0 = 80
CH1 = 80
CHMAX = max(CH0, CH1, 1)

N_ACC = 10240
RPT = N_ACC // NS


def _sc_mesh():
    return plsc.VectorSubcoreMesh(core_axis_name="c", subcore_axis_name="s")


def _deg_sc(dstp):
    @functools.partial(
        pl.kernel,
        mesh=_sc_mesh(),
        out_type=jax.ShapeDtypeStruct((NC * N_ACC, 16), jnp.float32),
        scratch_types=[
            pltpu.VMEM((NCH, C), jnp.int32),
            pltpu.VMEM((C, 16), jnp.float32),
            pltpu.VMEM_SHARED((N_ACC, 16), jnp.float32),
        ],
        compiler_params=pltpu.CompilerParams(use_tc_tiling_on_sc=False),
    )
    def k(dst_hbm, out_hbm, dst_v, ones_v, acc):
        c = lax.axis_index("c")
        s = lax.axis_index("s")
        w = c * NS + s

        pltpu.sync_copy(dst_hbm.at[pl.ds(w * NCH, NCH)], dst_v)

        def fill(i, val):
            ones_v[i, :] = jnp.full((16,), val, jnp.float32)
            return val

        lax.fori_loop(0, C, fill, 0.0)

        def zc(q, carry):
            pltpu.sync_copy(ones_v, acc.at[pl.ds(s * RPT + q * C, C)])
            return carry

        lax.fori_loop(0, RPT // C, zc, 0)
        lax.fori_loop(0, C, fill, 1.0)
        plsc.subcore_barrier()

        def body(j, carry):
            pltpu.sync_copy(ones_v, acc.at[dst_v.at[j]], add=True)
            return carry

        lax.fori_loop(0, NCH, body, 0)
        plsc.subcore_barrier()
        pltpu.sync_copy(
            acc.at[pl.ds(s * RPT, RPT)],
            out_hbm.at[pl.ds(c * N_ACC + s * RPT, RPT)],
        )

    return k(dstp)


def _msg_sc(y, srcp, dstp):
    @functools.partial(
        pl.kernel,
        mesh=_sc_mesh(),
        out_type=jax.ShapeDtypeStruct((2 * NC * N_ACC, H), jnp.float32),
        scratch_types=[
            pltpu.VMEM((CHMAX, C), jnp.int32),
            pltpu.VMEM((CHMAX, C), jnp.int32),
            [pltpu.VMEM((C, H), jnp.float32)] * NB,
            [pltpu.SemaphoreType.DMA] * NB,
            pltpu.VMEM_SHARED((N_ACC, H), jnp.float32),
        ],
        compiler_params=pltpu.CompilerParams(use_tc_tiling_on_sc=False),
    )
    def k(y_hbm, src_hbm, dst_hbm, out_hbm, src_v, dst_v, rows, sems, acc):
        c = lax.axis_index("c")
        s = lax.axis_index("s")

        def zb(t, carry):
            rows[0][t // 4, pl.ds((t % 4) * 16, 16)] = jnp.zeros(
                (16,), jnp.float32
            )
            return carry

        lax.fori_loop(0, C * (H // 16), zb, 0)

        def zc(q, carry):
            pltpu.sync_copy(rows[0], acc.at[pl.ds(s * RPT + q * C, C)])
            return carry

        lax.fori_loop(0, RPT // C, zc, 0)
        plsc.subcore_barrier()

        def run(cnt, base):
            if cnt == 0:
                return
            pltpu.sync_copy(
                src_hbm.at[pl.ds(base, cnt)], src_v.at[pl.ds(0, cnt)]
            )
            pltpu.sync_copy(
                dst_hbm.at[pl.ds(base, cnt)], dst_v.at[pl.ds(0, cnt)]
            )
            for b in range(NB):
                pltpu.async_copy(y_hbm.at[src_v.at[b]], rows[b], sems[b])

            def body(i, carry):
                for b in range(NB):
                    j = i * NB + b
                    pltpu.make_async_copy(
                        y_hbm.at[src_v.at[j]], rows[b], sems[b]
                    ).wait()
                    pltpu.sync_copy(rows[b], acc.at[dst_v.at[j]], add=True)

                    @pl.when(i < cnt // NB - 1)
                    def _():
                        pltpu.async_copy(
                            y_hbm.at[src_v.at[j + NB]], rows[b], sems[b]
                        )

                return carry

            lax.fori_loop(0, cnt // NB, body, 0)

        @pl.when(c == 0)
        def _():
            run(CH0, s * CH0)

        @pl.when(c == 1)
        def _():
            run(CH1, NS * CH0 + s * CH1)

        plsc.subcore_barrier()
        pltpu.sync_copy(
            acc.at[pl.ds(s * RPT, RPT)],
            out_hbm.at[pl.ds(c * N_ACC + s * RPT, RPT)],
        )
        pltpu.sync_copy(
            acc.at[pl.ds(s * RPT, RPT)],
            out_hbm.at[pl.ds((NC + c) * N_ACC + s * RPT, RPT)],
        )

    return k(y, srcp, dstp)


def _stage_a(dp0, dp1, x, w1):
    nb = 10
    rb = N // nb

    def body(d0, d1, xr, wr, dis_o, y_o):
        deg = d0[...] + d1[...] + 1.0
        dis = lax.rsqrt(deg)
        dis_o[...] = dis
        y_o[...] = (
            jnp.dot(xr[...], wr[...], preferred_element_type=jnp.float32)
            * dis[:, 0:1]
        )

    return pl.pallas_call(
        body,
        grid=(nb,),
        in_specs=[
            pl.BlockSpec((rb, 16), lambda j: (j, 0)),
            pl.BlockSpec((rb, 16), lambda j: (j, 0)),
            pl.BlockSpec((rb, D), lambda j: (j, 0)),
            pl.BlockSpec((D, H), lambda j: (0, 0)),
        ],
        out_specs=[
            pl.BlockSpec((rb, 16), lambda j: (j, 0)),
            pl.BlockSpec((rb, H), lambda j: (j, 0)),
        ],
        out_shape=[
            jax.ShapeDtypeStruct((N, 16), jnp.float32),
            jax.ShapeDtypeStruct((N, H), jnp.float32),
        ],
    )(dp0, dp1, x, w1)


def _stage_b(a0, a1, y1, dis, b1, w2):
    nb = 10
    rb = N // nb

    def body(a0r, a1r, yr, dr, br, wr, y2_o):
        dis = dr[...][:, 0:1]
        h = (a0r[...] + a1r[...] + yr[...]) * dis + br[...]
        h = jnp.maximum(h, 0.0)
        y2_o[...] = (
            jnp.dot(h, wr[...], preferred_element_type=jnp.float32) * dis
        )

    return pl.pallas_call(
        body,
        grid=(nb,),
        in_specs=[
            pl.BlockSpec((rb, H), lambda j: (j, 0)),
            pl.BlockSpec((rb, H), lambda j: (j, 0)),
            pl.BlockSpec((rb, H), lambda j: (j, 0)),
            pl.BlockSpec((rb, 16), lambda j: (j, 0)),
            pl.BlockSpec((1, H), lambda j: (0, 0)),
            pl.BlockSpec((H, H), lambda j: (0, 0)),
        ],
        out_specs=pl.BlockSpec((rb, H), lambda j: (j, 0)),
        out_shape=jax.ShapeDtypeStruct((N, H), jnp.float32),
    )(a0, a1, y1, dis, b1, w2)


def _stage_c(a0, a1, y2, dis, b2, batch3):
    nb = 10
    rb = N // nb

    def body(a0r, a1r, yr, dr, br, btr, out_o):
        j = pl.program_id(0)
        dis = dr[...][:, 0:1]
        h = (a0r[...] + a1r[...] + yr[...]) * dis + br[...]
        h = jnp.maximum(h, 0.0)
        bt = btr[0]
        ids = lax.broadcasted_iota(jnp.int32, (G, rb), 0)
        cmp = (ids == bt).astype(jnp.float32)
        hc = jnp.concatenate([h, jnp.ones((rb, H), jnp.float32)], axis=1)
        part = jnp.dot(cmp, hc, preferred_element_type=jnp.float32)

        @pl.when(j == 0)
        def _():
            out_o[...] = part

        @pl.when(j > 0)
        def _():
            out_o[...] = out_o[...] + part

        @pl.when(j == nb - 1)
        def _():
            res = out_o[...]
            sums = res[:, :H]
            cnt = jnp.maximum(res[:, H : H + 1], 1.0)
            out_o[...] = jnp.concatenate([sums / cnt, res[:, H:]], axis=1)

    return pl.pallas_call(
        body,
        grid=(nb,),
        in_specs=[
            pl.BlockSpec((rb, H), lambda j: (j, 0)),
            pl.BlockSpec((rb, H), lambda j: (j, 0)),
            pl.BlockSpec((rb, H), lambda j: (j, 0)),
            pl.BlockSpec((rb, 16), lambda j: (j, 0)),
            pl.BlockSpec((1, H), lambda j: (0, 0)),
            pl.BlockSpec((1, 1, rb), lambda j: (j, 0, 0)),
        ],
        out_specs=pl.BlockSpec((G, 2 * H), lambda j: (0, 0)),
        out_shape=jax.ShapeDtypeStruct((G, 2 * H), jnp.float32),
    )(a0, a1, y2, dis, b2, batch3)


def kernel(x, edge_index, batch, W1, b1, W2, b2):
    src = edge_index[0]
    dst = edge_index[1]
    e = src.shape[0]
    pad = E_PAD - e
    srcp = jnp.concatenate([src, jnp.zeros((pad,), jnp.int32)])
    pad_dst = N + jnp.arange(pad, dtype=jnp.int32) % (N_ACC - N)
    dstp = jnp.concatenate([dst, pad_dst])
    srcp = srcp.reshape(NW * NCH, C)
    dstp = dstp.reshape(NW * NCH, C)

    deg_parts = _deg_sc(dstp)
    dp0 = deg_parts[:N, :]
    dp1 = deg_parts[N_ACC : N_ACC + N, :]

    dis, y1 = _stage_a(dp0, dp1, x, W1)

    agg1 = _msg_sc(y1, srcp, dstp)
    y2 = _stage_b(
        agg1[:N, :], agg1[N_ACC : N_ACC + N, :], y1, dis,
        b1.reshape(1, H), W2,
    )

    agg2 = _msg_sc(y2, srcp, dstp)
    batch3 = batch.reshape(10, 1, N // 10)
    out = _stage_c(
        agg2[:N, :], agg2[N_ACC : N_ACC + N, :], y2, dis,
        b2.reshape(1, H), batch3,
    )
    return out[:, :H]


# speedup vs baseline: 1.0484x; 1.0484x over previous
import functools

import jax
import jax.numpy as jnp
from jax import lax
from jax.experimental import pallas as pl
from jax.experimental.pallas import tpu as pltpu
from jax.experimental.pallas import tpu_sc as plsc

N = 10000
D = 128
H = 64
G = 64

NC = 2
NS = 16
NW = NC * NS
C = 128
NCH = 80
EPT = NCH * C
E_PAD = EPT * NW
NB = 8

CH0 = 80
CH1 = 80
CHMAX = max(CH0, CH1, 1)

N_ACC = 10240
RPT = N_ACC // NS


def _sc_mesh():
    return plsc.VectorSubcoreMesh(core_axis_name="c", subcore_axis_name="s")


def _deg_sc(dstp):
    @functools.partial(
        pl.kernel,
        mesh=_sc_mesh(),
        out_type=jax.ShapeDtypeStruct((NC * N_ACC, 16), jnp.float32),
        scratch_types=[
            pltpu.VMEM((NCH, C), jnp.int32),
            pltpu.VMEM((C, 16), jnp.float32),
            pltpu.VMEM_SHARED((N_ACC, 16), jnp.float32),
        ],
        compiler_params=pltpu.CompilerParams(use_tc_tiling_on_sc=False),
    )
    def k(dst_hbm, out_hbm, dst_v, ones_v, acc):
        c = lax.axis_index("c")
        s = lax.axis_index("s")
        w = c * NS + s

        pltpu.sync_copy(dst_hbm.at[pl.ds(w * NCH, NCH)], dst_v)

        def fill(i, val):
            ones_v[i, :] = jnp.full((16,), val, jnp.float32)
            return val

        lax.fori_loop(0, C, fill, 0.0)

        def zc(q, carry):
            pltpu.sync_copy(ones_v, acc.at[pl.ds(s * RPT + q * C, C)])
            return carry

        lax.fori_loop(0, RPT // C, zc, 0)
        lax.fori_loop(0, C, fill, 1.0)
        plsc.subcore_barrier()

        def body(j, carry):
            pltpu.sync_copy(ones_v, acc.at[dst_v.at[j]], add=True)
            return carry

        lax.fori_loop(0, NCH, body, 0)
        plsc.subcore_barrier()
        pltpu.sync_copy(
            acc.at[pl.ds(s * RPT, RPT)],
            out_hbm.at[pl.ds(c * N_ACC + s * RPT, RPT)],
        )

    return k(dstp)


def _msg_sc(y, srcp, dstp):
    @functools.partial(
        pl.kernel,
        mesh=_sc_mesh(),
        out_type=jax.ShapeDtypeStruct((NC * N_ACC, H), jnp.float32),
        scratch_types=[
            pltpu.VMEM((CHMAX, C), jnp.int32),
            pltpu.VMEM((CHMAX, C), jnp.int32),
            [pltpu.VMEM((C, H), jnp.float32)] * NB,
            [pltpu.SemaphoreType.DMA] * NB,
            pltpu.VMEM_SHARED((N_ACC, H), jnp.float32),
        ],
        compiler_params=pltpu.CompilerParams(use_tc_tiling_on_sc=False),
    )
    def k(y_hbm, src_hbm, dst_hbm, out_hbm, src_v, dst_v, rows, sems, acc):
        c = lax.axis_index("c")
        s = lax.axis_index("s")

        def zb(t, carry):
            rows[0][t // 4, pl.ds((t % 4) * 16, 16)] = jnp.zeros(
                (16,), jnp.float32
            )
            return carry

        lax.fori_loop(0, C * (H // 16), zb, 0)

        def zc(q, carry):
            pltpu.sync_copy(rows[0], acc.at[pl.ds(s * RPT + q * C, C)])
            return carry

        lax.fori_loop(0, RPT // C, zc, 0)
        plsc.subcore_barrier()

        def run(cnt, base):
            if cnt == 0:
                return
            pltpu.sync_copy(
                src_hbm.at[pl.ds(base, cnt)], src_v.at[pl.ds(0, cnt)]
            )
            pltpu.sync_copy(
                dst_hbm.at[pl.ds(base, cnt)], dst_v.at[pl.ds(0, cnt)]
            )
            for b in range(NB):
                pltpu.async_copy(y_hbm.at[src_v.at[b]], rows[b], sems[b])

            def body(i, carry):
                for b in range(NB):
                    j = i * NB + b
                    pltpu.make_async_copy(
                        y_hbm.at[src_v.at[j]], rows[b], sems[b]
                    ).wait()
                    pltpu.sync_copy(rows[b], acc.at[dst_v.at[j]], add=True)

                    @pl.when(i < cnt // NB - 1)
                    def _():
                        pltpu.async_copy(
                            y_hbm.at[src_v.at[j + NB]], rows[b], sems[b]
                        )

                return carry

            lax.fori_loop(0, cnt // NB, body, 0)

        @pl.when(c == 0)
        def _():
            run(CH0, s * CH0)

        @pl.when(c == 1)
        def _():
            run(CH1, NS * CH0 + s * CH1)

        plsc.subcore_barrier()
        pltpu.sync_copy(
            acc.at[pl.ds(s * RPT, RPT)],
            out_hbm.at[pl.ds(c * N_ACC + s * RPT, RPT)],
        )

    return k(y, srcp, dstp)


def _stage_a(dp0, dp1, x, w1):
    nb = 10
    rb = N // nb

    def body(d0, d1, xr, wr, dis_o, y_o):
        deg = d0[...] + d1[...] + 1.0
        dis = lax.rsqrt(deg)
        dis_o[...] = dis
        y_o[...] = (
            jnp.dot(xr[...], wr[...], preferred_element_type=jnp.float32)
            * dis[:, 0:1]
        )

    return pl.pallas_call(
        body,
        grid=(nb,),
        in_specs=[
            pl.BlockSpec((rb, 16), lambda j: (j, 0)),
            pl.BlockSpec((rb, 16), lambda j: (j, 0)),
            pl.BlockSpec((rb, D), lambda j: (j, 0)),
            pl.BlockSpec((D, H), lambda j: (0, 0)),
        ],
        out_specs=[
            pl.BlockSpec((rb, 16), lambda j: (j, 0)),
            pl.BlockSpec((rb, H), lambda j: (j, 0)),
        ],
        out_shape=[
            jax.ShapeDtypeStruct((N, 16), jnp.float32),
            jax.ShapeDtypeStruct((N, H), jnp.float32),
        ],
    )(dp0, dp1, x, w1)


def _stage_b(a0, a1, y1, dis, b1, w2):
    nb = 10
    rb = N // nb

    def body(a0r, a1r, yr, dr, br, wr, y2_o):
        dis = dr[...][:, 0:1]
        h = (a0r[...] + a1r[...] + yr[...]) * dis + br[...]
        h = jnp.maximum(h, 0.0)
        y2_o[...] = (
            jnp.dot(h, wr[...], preferred_element_type=jnp.float32) * dis
        )

    return pl.pallas_call(
        body,
        grid=(nb,),
        in_specs=[
            pl.BlockSpec((rb, H), lambda j: (j, 0)),
            pl.BlockSpec((rb, H), lambda j: (j, 0)),
            pl.BlockSpec((rb, H), lambda j: (j, 0)),
            pl.BlockSpec((rb, 16), lambda j: (j, 0)),
            pl.BlockSpec((1, H), lambda j: (0, 0)),
            pl.BlockSpec((H, H), lambda j: (0, 0)),
        ],
        out_specs=pl.BlockSpec((rb, H), lambda j: (j, 0)),
        out_shape=jax.ShapeDtypeStruct((N, H), jnp.float32),
    )(a0, a1, y1, dis, b1, w2)


def _stage_c(a0, a1, y2, dis, b2, batch3):
    nb = 10
    rb = N // nb

    def body(a0r, a1r, yr, dr, br, btr, out_o):
        j = pl.program_id(0)
        dis = dr[...][:, 0:1]
        h = (a0r[...] + a1r[...] + yr[...]) * dis + br[...]
        h = jnp.maximum(h, 0.0)
        bt = btr[0]
        ids = lax.broadcasted_iota(jnp.int32, (G, rb), 0)
        cmp = (ids == bt).astype(jnp.float32)
        hc = jnp.concatenate([h, jnp.ones((rb, H), jnp.float32)], axis=1)
        part = jnp.dot(cmp, hc, preferred_element_type=jnp.float32)

        @pl.when(j == 0)
        def _():
            out_o[...] = part

        @pl.when(j > 0)
        def _():
            out_o[...] = out_o[...] + part

        @pl.when(j == nb - 1)
        def _():
            res = out_o[...]
            sums = res[:, :H]
            cnt = jnp.maximum(res[:, H : H + 1], 1.0)
            out_o[...] = jnp.concatenate([sums / cnt, res[:, H:]], axis=1)

    return pl.pallas_call(
        body,
        grid=(nb,),
        in_specs=[
            pl.BlockSpec((rb, H), lambda j: (j, 0)),
            pl.BlockSpec((rb, H), lambda j: (j, 0)),
            pl.BlockSpec((rb, H), lambda j: (j, 0)),
            pl.BlockSpec((rb, 16), lambda j: (j, 0)),
            pl.BlockSpec((1, H), lambda j: (0, 0)),
            pl.BlockSpec((1, 1, rb), lambda j: (j, 0, 0)),
        ],
        out_specs=pl.BlockSpec((G, 2 * H), lambda j: (0, 0)),
        out_shape=jax.ShapeDtypeStruct((G, 2 * H), jnp.float32),
    )(a0, a1, y2, dis, b2, batch3)


def kernel(x, edge_index, batch, W1, b1, W2, b2):
    src = edge_index[0]
    dst = edge_index[1]
    e = src.shape[0]
    pad = E_PAD - e
    srcp = jnp.concatenate([src, jnp.zeros((pad,), jnp.int32)])
    pad_dst = N + jnp.arange(pad, dtype=jnp.int32) % (N_ACC - N)
    dstp = jnp.concatenate([dst, pad_dst])
    srcp = srcp.reshape(NW * NCH, C)
    dstp = dstp.reshape(NW * NCH, C)

    deg_parts = _deg_sc(dstp)
    dp0 = deg_parts[:N, :]
    dp1 = deg_parts[N_ACC : N_ACC + N, :]

    dis, y1 = _stage_a(dp0, dp1, x, W1)

    agg1 = _msg_sc(y1, srcp, dstp)
    y2 = _stage_b(
        agg1[:N, :], agg1[N_ACC : N_ACC + N, :], y1, dis,
        b1.reshape(1, H), W2,
    )

    agg2 = _msg_sc(y2, srcp, dstp)
    batch3 = batch.reshape(10, 1, N // 10)
    out = _stage_c(
        agg2[:N, :], agg2[N_ACC : N_ACC + N, :], y2, dis,
        b2.reshape(1, H), batch3,
    )
    return out[:, :H]
